# Initial kernel scaffold; baseline (speedup 1.0000x reference)
import jax
import jax.numpy as jnp
from jax.experimental import pallas as pl


def kernel(x, edge_index, W1, a_src1, a_dst1, b1, W2, a_src2, a_dst2, b2):
    raise NotImplementedError("write your pallas kernel here")


# speedup vs baseline: 45.6214x; 45.6214x over previous
import functools

import jax
import jax.numpy as jnp
from jax import lax
from jax.experimental import pallas as pl
from jax.experimental.pallas import tpu as pltpu
from jax.experimental.pallas import tpu_sc as plsc

N = 4096
E = 65536
IN = 128
HID = 16
H = 4
OUT = 16
THRESH2 = 0.25
SLOPE = 0.2

CW = H * HID
L = 16
AW = CW + L
NC, NS = 2, 16
NW = NC * NS
EPW = E // NW
EB = 128
NBLK = EPW // EB
ROWS = N // NS
RD = 256
GRID = N // RD

_F32 = jnp.float32


def _head_expand():
    heads = lax.broadcasted_iota(jnp.int32, (H, CW), 0)
    chans = lax.broadcasted_iota(jnp.int32, (H, CW), 1)
    return jnp.where((chans // HID) == heads, 1.0, 0.0).astype(_F32)


def _proj_body(x_ref, w1_ref, as1_ref, ad1_ref,
               h_ref, asrc_ref, adst_ref, asrct_ref, adstt_ref, c_ref):
    h = jnp.dot(x_ref[...], w1_ref[...], preferred_element_type=_F32)
    h_ref[...] = h
    heads = lax.broadcasted_iota(jnp.int32, (H, CW), 0)
    chans = lax.broadcasted_iota(jnp.int32, (H, CW), 1)
    blk = (chans // HID) == heads
    a_s = jnp.where(blk, jnp.tile(as1_ref[...], (1, H)), 0.0)
    a_d = jnp.where(blk, jnp.tile(ad1_ref[...], (1, H)), 0.0)
    dn = (((1,), (1,)), ((), ()))
    asrc = lax.dot_general(h, a_s, dn, preferred_element_type=_F32)
    adst = lax.dot_general(h, a_d, dn, preferred_element_type=_F32)
    asrc_ref[...] = asrc
    adst_ref[...] = adst
    asrct_ref[...] = lax.dot_general(a_s, h, dn, preferred_element_type=_F32)
    adstt_ref[...] = lax.dot_general(a_d, h, dn, preferred_element_type=_F32)
    cm = (jnp.max(asrc, axis=0, keepdims=True)
          + jnp.max(adst, axis=0, keepdims=True))
    c = jnp.where(cm >= 0, cm, SLOPE * cm)
    c_ref[...] = jnp.concatenate([c, jnp.zeros((1, L - H), _F32)], axis=1)


def _proj_call(x, w1, as1, ad1):
    return pl.pallas_call(
        _proj_body,
        out_shape=(
            jax.ShapeDtypeStruct((N, CW), _F32),
            jax.ShapeDtypeStruct((N, H), _F32),
            jax.ShapeDtypeStruct((N, H), _F32),
            jax.ShapeDtypeStruct((H, N), _F32),
            jax.ShapeDtypeStruct((H, N), _F32),
            jax.ShapeDtypeStruct((1, L), _F32),
        ),
    )(x, w1, as1, ad1)


def _edge_body(src_h, dst_h, ast_h, adt_h, h_h, c_h,
               p0_h, p1_h,
               ast_v, adt_v, c_v, ids_s, ids_d, hrows, msg, acc):
    cid = lax.axis_index("c")
    sid = lax.axis_index("s")
    wid = sid * NC + cid

    pltpu.sync_copy(ast_h, ast_v)
    pltpu.sync_copy(adt_h, adt_v)
    pltpu.sync_copy(c_h, c_v)

    def _zero(i, carry):
        for k in range(AW // L):
            msg[i, pl.ds(k * L, L)] = jnp.zeros((L,), _F32)
        return carry
    lax.fori_loop(0, EB, _zero, 0)
    for t in range(ROWS // EB):
        pltpu.sync_copy(msg, acc.at[pl.ds(sid * ROWS + t * EB, EB)])
    plsc.subcore_barrier()

    cvec = c_v[...]
    cb = [cvec.at[jnp.full((L,), hh, jnp.int32)].get(mode="promise_in_bounds")
          for hh in range(H)]
    lane = jnp.arange(L, dtype=jnp.int32)
    base = wid * EPW

    def _block(b, carry):
        off = base + b * EB
        pltpu.sync_copy(src_h.at[pl.ds(off, EB)], ids_s)
        pltpu.sync_copy(dst_h.at[pl.ds(off, EB)], ids_d)
        pltpu.sync_copy(h_h.at[ids_s], hrows)

        def _group(g, carry2):
            sv = ids_s[pl.ds(g * L, L)]
            dv = ids_d[pl.ds(g * L, L)]
            ee = []
            for hh in range(H):
                off_h = jnp.int32(hh * N)
                e = (plsc.load_gather(ast_v, [sv + off_h])
                     + plsc.load_gather(adt_v, [dv + off_h]))
                e = jnp.where(e >= 0, e, SLOPE * e)
                ee.append(jnp.exp(e - cb[hh]))
            for j in range(L):
                row = g * L + j
                jv = jnp.full((L,), j, jnp.int32)
                den = jnp.zeros((L,), _F32)
                for hh in range(H):
                    bc = ee[hh].at[jv].get(mode="promise_in_bounds")
                    msg[row, pl.ds(hh * HID, HID)] = (
                        hrows[row, pl.ds(hh * HID, HID)] * bc)
                    den = jnp.where(lane == hh, bc, den)
                msg[row, pl.ds(CW, L)] = den
            return carry2
        lax.fori_loop(0, EB // L, _group, 0)
        pltpu.sync_copy(msg, acc.at[ids_d], add=True)
        return carry
    lax.fori_loop(0, NBLK, _block, 0)

    plsc.subcore_barrier()
    stripe = pl.ds(sid * ROWS, ROWS)

    @pl.when(cid == 0)
    def _():
        pltpu.sync_copy(acc.at[stripe], p0_h.at[stripe])

    @pl.when(cid == 1)
    def _():
        pltpu.sync_copy(acc.at[stripe], p1_h.at[stripe])


def _edge_call(src, dst, asrct, adstt, h, cvec):
    fn = pl.kernel(
        _edge_body,
        out_type=(
            jax.ShapeDtypeStruct((N, AW), _F32),
            jax.ShapeDtypeStruct((N, AW), _F32),
        ),
        mesh=plsc.VectorSubcoreMesh(core_axis_name="c", subcore_axis_name="s",
                                    num_cores=NC, num_subcores=NS),
        compiler_params=pltpu.CompilerParams(needs_layout_passes=False,
                                             use_tc_tiling_on_sc=False),
        scratch_types=[
            pltpu.VMEM((H * N,), _F32),
            pltpu.VMEM((H * N,), _F32),
            pltpu.VMEM((L,), _F32),
            pltpu.VMEM((EB,), jnp.int32),
            pltpu.VMEM((EB,), jnp.int32),
            pltpu.VMEM((EB, CW), _F32),
            pltpu.VMEM((EB, AW), _F32),
            pltpu.VMEM_SHARED((N, AW), _F32),
        ],
    )
    return fn(src, dst, asrct.reshape(H * N), adstt.reshape(H * N), h, cvec)


def _norm_body(p0_ref, p1_ref, h_ref, asrc_ref, adst_ref, c_ref, b1_ref,
               w2_ref, as2_ref, ad2_ref,
               h1_ref, hh_ref, sq_ref, gs_ref, gd_ref):
    c4 = c_ref[0:1, 0:H]
    es = asrc_ref[...] + adst_ref[...]
    es = jnp.where(es >= 0, es, SLOPE * es)
    eself = jnp.exp(es - c4)
    den4 = p0_ref[:, CW:CW + H] + p1_ref[:, CW:CW + H] + eself
    k4 = _head_expand()
    denw = jnp.dot(den4, k4, preferred_element_type=_F32)
    eselfw = jnp.dot(eself, k4, preferred_element_type=_F32)
    num = p0_ref[:, 0:CW] + p1_ref[:, 0:CW] + eselfw * h_ref[...]
    o = num / (denw + 1e-16) + b1_ref[...]
    h1 = jnp.where(o > 0, o, jnp.exp(jnp.minimum(o, 0.0)) - 1.0)
    h1_ref[...] = h1
    hh = jnp.dot(h1, w2_ref[...], preferred_element_type=_F32)
    hh_ref[...] = hh
    sq_ref[...] = jnp.sum(h1 * h1, axis=1, keepdims=True)
    gs_ref[...] = jnp.dot(hh, as2_ref[...], preferred_element_type=_F32)
    gd_ref[...] = jnp.dot(hh, ad2_ref[...], preferred_element_type=_F32)


def _norm_call(p0, p1, h, asrc, adst, cpad, b1, w2, as2, ad2):
    blk = lambda i: (i, 0)
    full = lambda i: (0, 0)
    return pl.pallas_call(
        _norm_body,
        grid=(GRID,),
        in_specs=[
            pl.BlockSpec((RD, AW), blk),
            pl.BlockSpec((RD, AW), blk),
            pl.BlockSpec((RD, CW), blk),
            pl.BlockSpec((RD, H), blk),
            pl.BlockSpec((RD, H), blk),
            pl.BlockSpec((1, L), full),
            pl.BlockSpec((1, CW), full),
            pl.BlockSpec((CW, OUT), full),
            pl.BlockSpec((OUT, 1), full),
            pl.BlockSpec((OUT, 1), full),
        ],
        out_specs=(
            pl.BlockSpec((RD, CW), blk),
            pl.BlockSpec((RD, OUT), blk),
            pl.BlockSpec((RD, 1), blk),
            pl.BlockSpec((RD, 1), blk),
            pl.BlockSpec((RD, 1), blk),
        ),
        out_shape=(
            jax.ShapeDtypeStruct((N, CW), _F32),
            jax.ShapeDtypeStruct((N, OUT), _F32),
            jax.ShapeDtypeStruct((N, 1), _F32),
            jax.ShapeDtypeStruct((N, 1), _F32),
            jax.ShapeDtypeStruct((N, 1), _F32),
        ),
    )(p0, p1, h, asrc, adst, cpad, b1, w2, as2, ad2)


def _dense_body(h1f_ref, h1b_ref, hh_ref, sqf_ref, sqb_ref, gs_ref, gd_ref,
                b2_ref, out_ref):
    ones_r = jnp.ones((RD, 1), _F32)
    ones_n = jnp.ones((N, 1), _F32)
    dn = (((1,), (1,)), ((), ()))
    a_mat = jnp.concatenate([sqb_ref[...], ones_r, -2.0 * h1b_ref[...]],
                            axis=1)
    b_mat = jnp.concatenate([ones_n, sqf_ref[...], h1f_ref[...]], axis=1)
    d2 = lax.dot_general(a_mat, b_mat, dn, preferred_element_type=_F32)
    ae = jnp.concatenate([gd_ref[...], ones_r], axis=1)
    be = jnp.concatenate([ones_n, gs_ref[...]], axis=1)
    er = lax.dot_general(ae, be, dn, preferred_element_type=_F32)
    e = jnp.where(er >= 0, er, SLOPE * er)
    e = jnp.where(d2 < THRESH2, e, -jnp.inf)
    m = jnp.max(e, axis=1, keepdims=True)
    p = jnp.exp(e - m)
    s = jnp.sum(p, axis=1, keepdims=True)
    o = jnp.dot(p, hh_ref[...], preferred_element_type=_F32) / s + b2_ref[...]
    z = o - jnp.max(o, axis=1, keepdims=True)
    out_ref[...] = z - jnp.log(jnp.sum(jnp.exp(z), axis=1, keepdims=True))


def _dense_call(h1, hh, sq, gs, gd, b2):
    blk = lambda i: (i, 0)
    full = lambda i: (0, 0)
    return pl.pallas_call(
        _dense_body,
        grid=(GRID,),
        in_specs=[
            pl.BlockSpec((N, CW), full),
            pl.BlockSpec((RD, CW), blk),
            pl.BlockSpec((N, OUT), full),
            pl.BlockSpec((N, 1), full),
            pl.BlockSpec((RD, 1), blk),
            pl.BlockSpec((N, 1), full),
            pl.BlockSpec((RD, 1), blk),
            pl.BlockSpec((1, OUT), full),
        ],
        out_specs=pl.BlockSpec((RD, OUT), blk),
        out_shape=jax.ShapeDtypeStruct((N, OUT), _F32),
    )(h1, h1, hh, sq, sq, gs, gd, b2)


@jax.jit
def kernel(x, edge_index, W1, a_src1, a_dst1, b1, W2, a_src2, a_dst2, b2):
    src = edge_index[0]
    dst = edge_index[1]
    h, asrc, adst, asrct, adstt, cpad = _proj_call(x, W1, a_src1, a_dst1)
    p0, p1 = _edge_call(src, dst, asrct, adstt, h, cpad.reshape(L))
    h1, hh, sq, gs, gd = _norm_call(
        p0, p1, h, asrc, adst, cpad, b1.reshape(1, CW), W2,
        a_src2.reshape(OUT, 1), a_dst2.reshape(OUT, 1))
    return _dense_call(h1, hh, sq, gs, gd, b2.reshape(1, OUT))


# speedup vs baseline: 52.0825x; 1.1416x over previous
import functools

import jax
import jax.numpy as jnp
from jax import lax
from jax.experimental import pallas as pl
from jax.experimental.pallas import tpu as pltpu
from jax.experimental.pallas import tpu_sc as plsc

N = 4096
E = 65536
IN = 128
HID = 16
H = 4
OUT = 16
THRESH2 = 0.25
SLOPE = 0.2

CW = H * HID
L = 16
AW = CW + L
NC, NS = 2, 16
NW = NC * NS
EPW = E // NW
EB = 128
NBLK = EPW // EB
ROWS = N // NS
RD = 256
GRID = N // RD

_F32 = jnp.float32


def _head_expand():
    heads = lax.broadcasted_iota(jnp.int32, (H, CW), 0)
    chans = lax.broadcasted_iota(jnp.int32, (H, CW), 1)
    return jnp.where((chans // HID) == heads, 1.0, 0.0).astype(_F32)


def _proj_body(x_ref, w1_ref, as1_ref, ad1_ref,
               h_ref, asrc_ref, adst_ref, asrct_ref, adstt_ref, c_ref):
    h = jnp.dot(x_ref[...], w1_ref[...], preferred_element_type=_F32)
    h_ref[...] = h
    heads = lax.broadcasted_iota(jnp.int32, (H, CW), 0)
    chans = lax.broadcasted_iota(jnp.int32, (H, CW), 1)
    blk = (chans // HID) == heads
    a_s = jnp.where(blk, jnp.tile(as1_ref[...], (1, H)), 0.0)
    a_d = jnp.where(blk, jnp.tile(ad1_ref[...], (1, H)), 0.0)
    dn = (((1,), (1,)), ((), ()))
    asrc = lax.dot_general(h, a_s, dn, preferred_element_type=_F32)
    adst = lax.dot_general(h, a_d, dn, preferred_element_type=_F32)
    asrc_ref[...] = asrc
    adst_ref[...] = adst
    asrct_ref[...] = lax.dot_general(a_s, h, dn, preferred_element_type=_F32)
    adstt_ref[...] = lax.dot_general(a_d, h, dn, preferred_element_type=_F32)
    cm = (jnp.max(asrc, axis=0, keepdims=True)
          + jnp.max(adst, axis=0, keepdims=True))
    c = jnp.where(cm >= 0, cm, SLOPE * cm)
    c_ref[...] = jnp.concatenate([c, jnp.zeros((1, L - H), _F32)], axis=1)


def _proj_call(x, w1, as1, ad1):
    return pl.pallas_call(
        _proj_body,
        out_shape=(
            jax.ShapeDtypeStruct((N, CW), _F32),
            jax.ShapeDtypeStruct((N, H), _F32),
            jax.ShapeDtypeStruct((N, H), _F32),
            jax.ShapeDtypeStruct((H, N), _F32),
            jax.ShapeDtypeStruct((H, N), _F32),
            jax.ShapeDtypeStruct((1, L), _F32),
        ),
    )(x, w1, as1, ad1)


def _edge_body(src_h, dst_h, ast_h, adt_h, h_h, c_h,
               p0_h, p1_h,
               ast_v, adt_v, c_v,
               ids_s0, ids_s1, ids_d0, ids_d1, sc_i0, sc_i1,
               hrows0, hrows1, msg0, msg1, acc,
               gsem, isem0, isem1, ssem0, ssem1):
    cid = lax.axis_index("c")
    sid = lax.axis_index("s")
    wid = sid * NC + cid
    bufs = [(ids_s0, ids_d0, sc_i0, hrows0, msg0, isem0, ssem0),
            (ids_s1, ids_d1, sc_i1, hrows1, msg1, isem1, ssem1)]

    pltpu.sync_copy(ast_h, ast_v)
    pltpu.sync_copy(adt_h, adt_v)
    pltpu.sync_copy(c_h, c_v)

    def _zero(i, carry):
        for k in range(AW // L):
            msg0[i, pl.ds(k * L, L)] = jnp.zeros((L,), _F32)
            msg1[i, pl.ds(k * L, L)] = jnp.zeros((L,), _F32)
        return carry
    lax.fori_loop(0, EB, _zero, 0)
    for t in range(ROWS // EB):
        pltpu.sync_copy(msg0, acc.at[pl.ds(sid * ROWS + t * EB, EB)])
    plsc.subcore_barrier()

    cvec = c_v[...]
    cb = [cvec.at[jnp.full((L,), hh, jnp.int32)].get(mode="promise_in_bounds")
          for hh in range(H)]
    lane = jnp.arange(L, dtype=jnp.int32)
    base = wid * EPW

    def _compute(ids_sr, ids_dr, hrowsr, msgr, sc_ir):
        for k in range(EB // L):
            sc_ir[pl.ds(k * L, L)] = ids_dr[pl.ds(k * L, L)]

        def _group(g, carry2):
            sv = ids_sr[pl.ds(g * L, L)]
            dv = ids_dr[pl.ds(g * L, L)]
            ee = []
            for hh in range(H):
                off_h = jnp.int32(hh * N)
                e = (plsc.load_gather(ast_v, [sv + off_h])
                     + plsc.load_gather(adt_v, [dv + off_h]))
                e = jnp.where(e >= 0, e, SLOPE * e)
                ee.append(jnp.exp(e - cb[hh]))
            row0 = g * L
            for hh in range(H):
                plsc.store_scatter(
                    msgr, [row0 + lane, jnp.full((L,), CW + hh, jnp.int32)],
                    ee[hh])
            for j in range(L):
                row = row0 + j
                jv = jnp.full((L,), j, jnp.int32)
                for hh in range(H):
                    bc = ee[hh].at[jv].get(mode="promise_in_bounds")
                    msgr[row, pl.ds(hh * HID, HID)] = (
                        hrowsr[row, pl.ds(hh * HID, HID)] * bc)
            return carry2
        lax.fori_loop(0, EB // L, _group, 0)

    pltpu.sync_copy(src_h.at[pl.ds(base, EB)], ids_s0)
    pltpu.sync_copy(dst_h.at[pl.ds(base, EB)], ids_d0)
    pltpu.async_copy(h_h.at[ids_s0], hrows0, gsem)
    pltpu.async_copy(src_h.at[pl.ds(base + EB, EB)], ids_s1, isem1)
    pltpu.async_copy(dst_h.at[pl.ds(base + EB, EB)], ids_d1, isem1)

    def _half(x, b, i):
        ids_sr, ids_dr, sc_ir, hrowsr, msgr, isem, ssem = bufs[x]
        o_ids_sr, o_ids_dr, _, o_hrowsr, _, o_isem, _ = bufs[1 - x]
        pltpu.make_async_copy(h_h.at[ids_sr], hrowsr, gsem).wait()

        @pl.when(i > 0)
        def _():
            pltpu.make_async_copy(msgr, acc.at[sc_ir], ssem).wait()

        def _next_gather():
            pltpu.make_async_copy(src_h.at[pl.ds(0, EB)], o_ids_sr,
                                  o_isem).wait()
            pltpu.make_async_copy(dst_h.at[pl.ds(0, EB)], o_ids_dr,
                                  o_isem).wait()
            pltpu.async_copy(h_h.at[o_ids_sr], o_hrowsr, gsem)
        if x == 0:
            _next_gather()
        else:
            pl.when(i < NBLK // 2 - 1)(_next_gather)

        _compute(ids_sr, ids_dr, hrowsr, msgr, sc_ir)
        pltpu.async_copy(msgr, acc.at[sc_ir], ssem, add=True)

        @pl.when(i < NBLK // 2 - 1)
        def _():
            off2 = base + (b + 2) * EB
            pltpu.async_copy(src_h.at[pl.ds(off2, EB)], ids_sr, isem)
            pltpu.async_copy(dst_h.at[pl.ds(off2, EB)], ids_dr, isem)

    def _body(i, carry):
        _half(0, 2 * i, i)
        _half(1, 2 * i + 1, i)
        return carry
    lax.fori_loop(0, NBLK // 2, _body, 0)

    pltpu.make_async_copy(msg0, acc.at[sc_i0], ssem0).wait()
    pltpu.make_async_copy(msg1, acc.at[sc_i1], ssem1).wait()

    plsc.subcore_barrier()
    stripe = pl.ds(sid * ROWS, ROWS)

    @pl.when(cid == 0)
    def _():
        pltpu.sync_copy(acc.at[stripe], p0_h.at[stripe])

    @pl.when(cid == 1)
    def _():
        pltpu.sync_copy(acc.at[stripe], p1_h.at[stripe])


def _edge_call(src, dst, asrct, adstt, h, cvec):
    fn = pl.kernel(
        _edge_body,
        out_type=(
            jax.ShapeDtypeStruct((N, AW), _F32),
            jax.ShapeDtypeStruct((N, AW), _F32),
        ),
        mesh=plsc.VectorSubcoreMesh(core_axis_name="c", subcore_axis_name="s",
                                    num_cores=NC, num_subcores=NS),
        compiler_params=pltpu.CompilerParams(needs_layout_passes=False,
                                             use_tc_tiling_on_sc=False),
        scratch_types=[
            pltpu.VMEM((H * N,), _F32),
            pltpu.VMEM((H * N,), _F32),
            pltpu.VMEM((L,), _F32),
            pltpu.VMEM((EB,), jnp.int32),
            pltpu.VMEM((EB,), jnp.int32),
            pltpu.VMEM((EB,), jnp.int32),
            pltpu.VMEM((EB,), jnp.int32),
            pltpu.VMEM((EB,), jnp.int32),
            pltpu.VMEM((EB,), jnp.int32),
            pltpu.VMEM((EB, CW), _F32),
            pltpu.VMEM((EB, CW), _F32),
            pltpu.VMEM((EB, AW), _F32),
            pltpu.VMEM((EB, AW), _F32),
            pltpu.VMEM_SHARED((N, AW), _F32),
            pltpu.SemaphoreType.DMA,
            pltpu.SemaphoreType.DMA,
            pltpu.SemaphoreType.DMA,
            pltpu.SemaphoreType.DMA,
            pltpu.SemaphoreType.DMA,
        ],
    )
    return fn(src, dst, asrct.reshape(H * N), adstt.reshape(H * N), h, cvec)


def _norm_body(p0_ref, p1_ref, h_ref, asrc_ref, adst_ref, c_ref, b1_ref,
               w2_ref, as2_ref, ad2_ref,
               h1_ref, hh_ref, sq_ref, gs_ref, gd_ref):
    c4 = c_ref[0:1, 0:H]
    es = asrc_ref[...] + adst_ref[...]
    es = jnp.where(es >= 0, es, SLOPE * es)
    eself = jnp.exp(es - c4)
    den4 = p0_ref[:, CW:CW + H] + p1_ref[:, CW:CW + H] + eself
    k4 = _head_expand()
    denw = jnp.dot(den4, k4, preferred_element_type=_F32)
    eselfw = jnp.dot(eself, k4, preferred_element_type=_F32)
    num = p0_ref[:, 0:CW] + p1_ref[:, 0:CW] + eselfw * h_ref[...]
    o = num / (denw + 1e-16) + b1_ref[...]
    h1 = jnp.where(o > 0, o, jnp.exp(jnp.minimum(o, 0.0)) - 1.0)
    h1_ref[...] = h1
    hh = jnp.dot(h1, w2_ref[...], preferred_element_type=_F32)
    hh_ref[...] = hh
    sq_ref[...] = jnp.sum(h1 * h1, axis=1, keepdims=True)
    gs_ref[...] = jnp.dot(hh, as2_ref[...], preferred_element_type=_F32)
    gd_ref[...] = jnp.dot(hh, ad2_ref[...], preferred_element_type=_F32)


def _norm_call(p0, p1, h, asrc, adst, cpad, b1, w2, as2, ad2):
    blk = lambda i: (i, 0)
    full = lambda i: (0, 0)
    return pl.pallas_call(
        _norm_body,
        grid=(GRID,),
        in_specs=[
            pl.BlockSpec((RD, AW), blk),
            pl.BlockSpec((RD, AW), blk),
            pl.BlockSpec((RD, CW), blk),
            pl.BlockSpec((RD, H), blk),
            pl.BlockSpec((RD, H), blk),
            pl.BlockSpec((1, L), full),
            pl.BlockSpec((1, CW), full),
            pl.BlockSpec((CW, OUT), full),
            pl.BlockSpec((OUT, 1), full),
            pl.BlockSpec((OUT, 1), full),
        ],
        out_specs=(
            pl.BlockSpec((RD, CW), blk),
            pl.BlockSpec((RD, OUT), blk),
            pl.BlockSpec((RD, 1), blk),
            pl.BlockSpec((RD, 1), blk),
            pl.BlockSpec((RD, 1), blk),
        ),
        out_shape=(
            jax.ShapeDtypeStruct((N, CW), _F32),
            jax.ShapeDtypeStruct((N, OUT), _F32),
            jax.ShapeDtypeStruct((N, 1), _F32),
            jax.ShapeDtypeStruct((N, 1), _F32),
            jax.ShapeDtypeStruct((N, 1), _F32),
        ),
    )(p0, p1, h, asrc, adst, cpad, b1, w2, as2, ad2)


def _dense_body(h1f_ref, h1b_ref, hh_ref, sqf_ref, sqb_ref, gs_ref, gd_ref,
                b2_ref, out_ref):
    ones_r = jnp.ones((RD, 1), _F32)
    ones_n = jnp.ones((N, 1), _F32)
    dn = (((1,), (1,)), ((), ()))
    a_mat = jnp.concatenate([sqb_ref[...], ones_r, -2.0 * h1b_ref[...]],
                            axis=1)
    b_mat = jnp.concatenate([ones_n, sqf_ref[...], h1f_ref[...]], axis=1)
    d2 = lax.dot_general(a_mat, b_mat, dn, preferred_element_type=_F32)
    ae = jnp.concatenate([gd_ref[...], ones_r], axis=1)
    be = jnp.concatenate([ones_n, gs_ref[...]], axis=1)
    er = lax.dot_general(ae, be, dn, preferred_element_type=_F32)
    e = jnp.where(er >= 0, er, SLOPE * er)
    e = jnp.where(d2 < THRESH2, e, -jnp.inf)
    m = jnp.max(e, axis=1, keepdims=True)
    p = jnp.exp(e - m)
    s = jnp.sum(p, axis=1, keepdims=True)
    o = jnp.dot(p, hh_ref[...], preferred_element_type=_F32) / s + b2_ref[...]
    z = o - jnp.max(o, axis=1, keepdims=True)
    out_ref[...] = z - jnp.log(jnp.sum(jnp.exp(z), axis=1, keepdims=True))


def _dense_call(h1, hh, sq, gs, gd, b2):
    blk = lambda i: (i, 0)
    full = lambda i: (0, 0)
    return pl.pallas_call(
        _dense_body,
        grid=(GRID,),
        in_specs=[
            pl.BlockSpec((N, CW), full),
            pl.BlockSpec((RD, CW), blk),
            pl.BlockSpec((N, OUT), full),
            pl.BlockSpec((N, 1), full),
            pl.BlockSpec((RD, 1), blk),
            pl.BlockSpec((N, 1), full),
            pl.BlockSpec((RD, 1), blk),
            pl.BlockSpec((1, OUT), full),
        ],
        out_specs=pl.BlockSpec((RD, OUT), blk),
        out_shape=jax.ShapeDtypeStruct((N, OUT), _F32),
    )(h1, h1, hh, sq, sq, gs, gd, b2)


@jax.jit
def kernel(x, edge_index, W1, a_src1, a_dst1, b1, W2, a_src2, a_dst2, b2):
    src = edge_index[0]
    dst = edge_index[1]
    h, asrc, adst, asrct, adstt, cpad = _proj_call(x, W1, a_src1, a_dst1)
    p0, p1 = _edge_call(src, dst, asrct, adstt, h, cpad.reshape(L))
    h1, hh, sq, gs, gd = _norm_call(
        p0, p1, h, asrc, adst, cpad, b1.reshape(1, CW), W2,
        a_src2.reshape(OUT, 1), a_dst2.reshape(OUT, 1))
    return _dense_call(h1, hh, sq, gs, gd, b2.reshape(1, OUT))


# speedup vs baseline: 55.4526x; 1.0647x over previous
import functools

import jax
import jax.numpy as jnp
from jax import lax
from jax.experimental import pallas as pl
from jax.experimental.pallas import tpu as pltpu
from jax.experimental.pallas import tpu_sc as plsc

N = 4096
E = 65536
IN = 128
HID = 16
H = 4
OUT = 16
THRESH2 = 0.25
SLOPE = 0.2

CW = H * HID
L = 16
AW = CW + L
NC, NS = 2, 16
NW = NC * NS
EPW = E // NW
EB = 128
NBLK = EPW // EB
ROWS = N // NS
RD = 256
GRID = N // RD

_F32 = jnp.float32


def _head_expand():
    heads = lax.broadcasted_iota(jnp.int32, (H, CW), 0)
    chans = lax.broadcasted_iota(jnp.int32, (H, CW), 1)
    return jnp.where((chans // HID) == heads, 1.0, 0.0).astype(_F32)


def _proj_body(x_ref, w1_ref, as1_ref, ad1_ref,
               h_ref, asrc_ref, adst_ref, asrct_ref, adstt_ref, c_ref):
    h = jnp.dot(x_ref[...], w1_ref[...], preferred_element_type=_F32)
    h_ref[...] = h
    heads = lax.broadcasted_iota(jnp.int32, (H, CW), 0)
    chans = lax.broadcasted_iota(jnp.int32, (H, CW), 1)
    blk = (chans // HID) == heads
    a_s = jnp.where(blk, jnp.tile(as1_ref[...], (1, H)), 0.0)
    a_d = jnp.where(blk, jnp.tile(ad1_ref[...], (1, H)), 0.0)
    dn = (((1,), (1,)), ((), ()))
    asrc = lax.dot_general(h, a_s, dn, preferred_element_type=_F32)
    adst = lax.dot_general(h, a_d, dn, preferred_element_type=_F32)
    asrc_ref[...] = asrc
    adst_ref[...] = adst
    asrct_ref[...] = lax.dot_general(a_s, h, dn, preferred_element_type=_F32)
    adstt_ref[...] = lax.dot_general(a_d, h, dn, preferred_element_type=_F32)
    cm = (jnp.max(asrc, axis=0, keepdims=True)
          + jnp.max(adst, axis=0, keepdims=True))
    c = jnp.where(cm >= 0, cm, SLOPE * cm)
    c_ref[...] = jnp.concatenate([c, jnp.zeros((1, L - H), _F32)], axis=1)


def _proj_call(x, w1, as1, ad1):
    return pl.pallas_call(
        _proj_body,
        out_shape=(
            jax.ShapeDtypeStruct((N, CW), _F32),
            jax.ShapeDtypeStruct((N, H), _F32),
            jax.ShapeDtypeStruct((N, H), _F32),
            jax.ShapeDtypeStruct((H, N), _F32),
            jax.ShapeDtypeStruct((H, N), _F32),
            jax.ShapeDtypeStruct((1, L), _F32),
        ),
    )(x, w1, as1, ad1)


def _edge_body(src_h, dst_h, ast_h, adt_h, h_h, c_h,
               p0_h, p1_h,
               ast_v, adt_v, c_v,
               ids_s0, ids_s1, ids_d0, ids_d1, sc_i0, sc_i1,
               hrows0, hrows1, msg0, msg1, acc,
               gsem, isem0, isem1, ssem0, ssem1):
    cid = lax.axis_index("c")
    sid = lax.axis_index("s")
    wid = sid * NC + cid
    bufs = [(ids_s0, ids_d0, sc_i0, hrows0, msg0, isem0, ssem0),
            (ids_s1, ids_d1, sc_i1, hrows1, msg1, isem1, ssem1)]

    pltpu.sync_copy(ast_h, ast_v)
    pltpu.sync_copy(adt_h, adt_v)
    pltpu.sync_copy(c_h, c_v)

    def _zero(i, carry):
        for k in range(AW // L):
            msg0[i, pl.ds(k * L, L)] = jnp.zeros((L,), _F32)
            msg1[i, pl.ds(k * L, L)] = jnp.zeros((L,), _F32)
        return carry
    lax.fori_loop(0, EB, _zero, 0)
    for t in range(ROWS // EB):
        pltpu.sync_copy(msg0, acc.at[pl.ds(sid * ROWS + t * EB, EB)])
    plsc.subcore_barrier()

    cvec = c_v[...]
    cb = [cvec.at[jnp.full((L,), hh, jnp.int32)].get(mode="promise_in_bounds")
          for hh in range(H)]
    lane = jnp.arange(L, dtype=jnp.int32)
    base = wid * EPW

    def _compute(ids_sr, ids_dr, hrowsr, msgr, sc_ir):
        for k in range(EB // L):
            sc_ir[pl.ds(k * L, L)] = ids_dr[pl.ds(k * L, L)]

        def _group(g, carry2):
            sv = ids_sr[pl.ds(g * L, L)]
            dv = ids_dr[pl.ds(g * L, L)]
            ee = []
            for hh in range(H):
                off_h = jnp.int32(hh * N)
                e = (plsc.load_gather(ast_v, [sv + off_h])
                     + plsc.load_gather(adt_v, [dv + off_h]))
                e = jnp.where(e >= 0, e, SLOPE * e)
                ee.append(jnp.exp(e - cb[hh]))
            row0 = g * L
            for hh in range(H):
                plsc.store_scatter(
                    msgr, [row0 + lane, jnp.full((L,), CW + hh, jnp.int32)],
                    ee[hh])
            for j in range(L):
                row = row0 + j
                jv = jnp.full((L,), j, jnp.int32)
                for hh in range(H):
                    bc = ee[hh].at[jv].get(mode="promise_in_bounds")
                    msgr[row, pl.ds(hh * HID, HID)] = (
                        hrowsr[row, pl.ds(hh * HID, HID)] * bc)
            return carry2
        lax.fori_loop(0, EB // L, _group, 0)

    pltpu.sync_copy(src_h.at[pl.ds(base, EB)], ids_s0)
    pltpu.sync_copy(dst_h.at[pl.ds(base, EB)], ids_d0)
    pltpu.async_copy(h_h.at[ids_s0], hrows0, gsem)
    pltpu.async_copy(src_h.at[pl.ds(base + EB, EB)], ids_s1, isem1)
    pltpu.async_copy(dst_h.at[pl.ds(base + EB, EB)], ids_d1, isem1)

    def _half(x, b, i):
        ids_sr, ids_dr, sc_ir, hrowsr, msgr, isem, ssem = bufs[x]
        o_ids_sr, o_ids_dr, _, o_hrowsr, _, o_isem, _ = bufs[1 - x]
        pltpu.make_async_copy(h_h.at[ids_sr], hrowsr, gsem).wait()

        @pl.when(i > 0)
        def _():
            pltpu.make_async_copy(msgr, acc.at[sc_ir], ssem).wait()

        def _next_gather():
            pltpu.make_async_copy(src_h.at[pl.ds(0, EB)], o_ids_sr,
                                  o_isem).wait()
            pltpu.make_async_copy(dst_h.at[pl.ds(0, EB)], o_ids_dr,
                                  o_isem).wait()
            pltpu.async_copy(h_h.at[o_ids_sr], o_hrowsr, gsem)

        if x == 0:
            _next_gather()
        else:
            pl.when(i < NBLK // 2 - 1)(_next_gather)

        _compute(ids_sr, ids_dr, hrowsr, msgr, sc_ir)
        pltpu.async_copy(msgr, acc.at[sc_ir], ssem, add=True)

        @pl.when(i < NBLK // 2 - 1)
        def _():
            off2 = base + (b + 2) * EB
            pltpu.async_copy(src_h.at[pl.ds(off2, EB)], ids_sr, isem)
            pltpu.async_copy(dst_h.at[pl.ds(off2, EB)], ids_dr, isem)

    def _body(i, carry):
        _half(0, 2 * i, i)
        _half(1, 2 * i + 1, i)
        return carry
    lax.fori_loop(0, NBLK // 2, _body, 0)

    pltpu.make_async_copy(msg0, acc.at[sc_i0], ssem0).wait()
    pltpu.make_async_copy(msg1, acc.at[sc_i1], ssem1).wait()


    plsc.subcore_barrier()
    stripe = pl.ds(sid * ROWS, ROWS)

    @pl.when(cid == 0)
    def _():
        pltpu.sync_copy(acc.at[stripe], p0_h.at[stripe])

    @pl.when(cid == 1)
    def _():
        pltpu.sync_copy(acc.at[stripe], p1_h.at[stripe])


def _edge_call(src, dst, asrct, adstt, h, cvec):
    fn = pl.kernel(
        _edge_body,
        out_type=(
            jax.ShapeDtypeStruct((N, AW), _F32),
            jax.ShapeDtypeStruct((N, AW), _F32),
        ),
        mesh=plsc.VectorSubcoreMesh(core_axis_name="c", subcore_axis_name="s",
                                    num_cores=NC, num_subcores=NS),
        compiler_params=pltpu.CompilerParams(needs_layout_passes=False,
                                             use_tc_tiling_on_sc=False),
        scratch_types=[
            pltpu.VMEM((H * N,), _F32),
            pltpu.VMEM((H * N,), _F32),
            pltpu.VMEM((L,), _F32),
            pltpu.VMEM((EB,), jnp.int32),
            pltpu.VMEM((EB,), jnp.int32),
            pltpu.VMEM((EB,), jnp.int32),
            pltpu.VMEM((EB,), jnp.int32),
            pltpu.VMEM((EB,), jnp.int32),
            pltpu.VMEM((EB,), jnp.int32),
            pltpu.VMEM((EB, CW), _F32),
            pltpu.VMEM((EB, CW), _F32),
            pltpu.VMEM((EB, AW), _F32),
            pltpu.VMEM((EB, AW), _F32),
            pltpu.VMEM_SHARED((N, AW), _F32),
            pltpu.SemaphoreType.DMA,
            pltpu.SemaphoreType.DMA,
            pltpu.SemaphoreType.DMA,
            pltpu.SemaphoreType.DMA,
            pltpu.SemaphoreType.DMA,
        ],
    )
    return fn(src, dst, asrct.reshape(H * N), adstt.reshape(H * N), h, cvec)


def _norm_body(p0_ref, p1_ref, h_ref, asrc_ref, adst_ref, c_ref, b1_ref,
               w2_ref, as2_ref, ad2_ref,
               amat_ref, bmat_ref, ae_ref, be_ref, hh_ref, mstat_ref):
    i = pl.program_id(0)
    c4 = c_ref[0:1, 0:H]
    es = asrc_ref[...] + adst_ref[...]
    es = jnp.where(es >= 0, es, SLOPE * es)
    eself = jnp.exp(es - c4)
    den4 = p0_ref[:, CW:CW + H] + p1_ref[:, CW:CW + H] + eself
    k4 = _head_expand()
    denw = jnp.dot(den4, k4, preferred_element_type=_F32)
    eselfw = jnp.dot(eself, k4, preferred_element_type=_F32)
    num = p0_ref[:, 0:CW] + p1_ref[:, 0:CW] + eselfw * h_ref[...]
    o = num / (denw + 1e-16) + b1_ref[...]
    h1 = jnp.where(o > 0, o, jnp.exp(jnp.minimum(o, 0.0)) - 1.0)
    hh = jnp.dot(h1, w2_ref[...], preferred_element_type=_F32)
    hh_ref[...] = hh
    sq = jnp.sum(h1 * h1, axis=1, keepdims=True)
    gs = jnp.dot(hh, as2_ref[...], preferred_element_type=_F32)
    gd = jnp.dot(hh, ad2_ref[...], preferred_element_type=_F32)
    ones_r = jnp.ones((RD, 1), _F32)
    amat_ref[...] = jnp.concatenate([sq, ones_r, -2.0 * h1], axis=1)
    bmat_ref[...] = jnp.concatenate([ones_r, sq, h1], axis=1)
    ae_ref[...] = jnp.concatenate([gd, ones_r], axis=1)
    be_ref[...] = jnp.concatenate([ones_r, gs], axis=1)
    bm = jnp.concatenate([jnp.max(gd, axis=0, keepdims=True),
                          jnp.max(gs, axis=0, keepdims=True)], axis=1)

    @pl.when(i == 0)
    def _():
        mstat_ref[...] = bm

    @pl.when(i > 0)
    def _():
        mstat_ref[...] = jnp.maximum(mstat_ref[...], bm)


def _norm_call(p0, p1, h, asrc, adst, cpad, b1, w2, as2, ad2):
    blk = lambda i: (i, 0)
    full = lambda i: (0, 0)
    return pl.pallas_call(
        _norm_body,
        grid=(GRID,),
        in_specs=[
            pl.BlockSpec((RD, AW), blk),
            pl.BlockSpec((RD, AW), blk),
            pl.BlockSpec((RD, CW), blk),
            pl.BlockSpec((RD, H), blk),
            pl.BlockSpec((RD, H), blk),
            pl.BlockSpec((1, L), full),
            pl.BlockSpec((1, CW), full),
            pl.BlockSpec((CW, OUT), full),
            pl.BlockSpec((OUT, 1), full),
            pl.BlockSpec((OUT, 1), full),
        ],
        out_specs=(
            pl.BlockSpec((RD, CW + 2), blk),
            pl.BlockSpec((RD, CW + 2), blk),
            pl.BlockSpec((RD, 2), blk),
            pl.BlockSpec((RD, 2), blk),
            pl.BlockSpec((RD, OUT), blk),
            pl.BlockSpec((1, 2), full),
        ),
        out_shape=(
            jax.ShapeDtypeStruct((N, CW + 2), _F32),
            jax.ShapeDtypeStruct((N, CW + 2), _F32),
            jax.ShapeDtypeStruct((N, 2), _F32),
            jax.ShapeDtypeStruct((N, 2), _F32),
            jax.ShapeDtypeStruct((N, OUT), _F32),
            jax.ShapeDtypeStruct((1, 2), _F32),
        ),
    )(p0, p1, h, asrc, adst, cpad, b1, w2, as2, ad2)


def _dense_body(amat_ref, bmat_ref, ae_ref, be_ref, hh_ref, mstat_ref,
                b2_ref, out_ref):
    dn = (((1,), (1,)), ((), ()))
    d2 = lax.dot_general(amat_ref[...], bmat_ref[...], dn,
                         preferred_element_type=_F32)
    er = lax.dot_general(ae_ref[...], be_ref[...], dn,
                         preferred_element_type=_F32)
    mbq = mstat_ref[0, 0] + mstat_ref[0, 1]
    mb = jnp.where(mbq >= 0, mbq, SLOPE * mbq)
    e = jnp.maximum(er, SLOPE * er)
    p = jnp.where(d2 < THRESH2, jnp.exp(e - mb), 0.0)
    s = jnp.sum(p, axis=1, keepdims=True)
    o = (jnp.dot(p, hh_ref[...], preferred_element_type=_F32) / s
         + b2_ref[...])
    z = o - jnp.max(o, axis=1, keepdims=True)
    out_ref[...] = z - jnp.log(jnp.sum(jnp.exp(z), axis=1, keepdims=True))


def _dense_call(amat, bmat, ae, be, hh, mstat, b2):
    blk = lambda i: (i, 0)
    full = lambda i: (0, 0)
    return pl.pallas_call(
        _dense_body,
        grid=(GRID,),
        in_specs=[
            pl.BlockSpec((RD, CW + 2), blk),
            pl.BlockSpec((N, CW + 2), full),
            pl.BlockSpec((RD, 2), blk),
            pl.BlockSpec((N, 2), full),
            pl.BlockSpec((N, OUT), full),
            pl.BlockSpec((1, 2), full),
            pl.BlockSpec((1, OUT), full),
        ],
        out_specs=pl.BlockSpec((RD, OUT), blk),
        out_shape=jax.ShapeDtypeStruct((N, OUT), _F32),
    )(amat, bmat, ae, be, hh, mstat, b2)


@jax.jit
def kernel(x, edge_index, W1, a_src1, a_dst1, b1, W2, a_src2, a_dst2, b2):
    src = edge_index[0]
    dst = edge_index[1]
    h, asrc, adst, asrct, adstt, cpad = _proj_call(x, W1, a_src1, a_dst1)
    p0, p1 = _edge_call(src, dst, asrct, adstt, h, cpad.reshape(L))
    amat, bmat, ae, be, hh, mstat = _norm_call(
        p0, p1, h, asrc, adst, cpad, b1.reshape(1, CW), W2,
        a_src2.reshape(OUT, 1), a_dst2.reshape(OUT, 1))
    return _dense_call(amat, bmat, ae, be, hh, mstat, b2.reshape(1, OUT))


# speedup vs baseline: 60.9940x; 1.0999x over previous
import functools

import jax
import jax.numpy as jnp
from jax import lax
from jax.experimental import pallas as pl
from jax.experimental.pallas import tpu as pltpu
from jax.experimental.pallas import tpu_sc as plsc

N = 4096
E = 65536
IN = 128
HID = 16
H = 4
OUT = 16
THRESH2 = 0.25
SLOPE = 0.2

CW = H * HID
L = 16
AW = CW + L
NC, NS = 2, 16
NW = NC * NS
EPW = E // NW
EB = 128
NBLK = EPW // EB
ROWS = N // NS
RD = 256
GRID = N // RD

_F32 = jnp.float32


def _head_expand():
    heads = lax.broadcasted_iota(jnp.int32, (H, CW), 0)
    chans = lax.broadcasted_iota(jnp.int32, (H, CW), 1)
    return jnp.where((chans // HID) == heads, 1.0, 0.0).astype(_F32)


def _proj_body(x_ref, w1_ref, as1_ref, ad1_ref,
               h_ref, asrc_ref, adst_ref, asrct_ref, adstt_ref, c_ref):
    h = jnp.dot(x_ref[...], w1_ref[...], preferred_element_type=_F32)
    h_ref[...] = h
    heads = lax.broadcasted_iota(jnp.int32, (H, CW), 0)
    chans = lax.broadcasted_iota(jnp.int32, (H, CW), 1)
    blk = (chans // HID) == heads
    a_s = jnp.where(blk, jnp.tile(as1_ref[...], (1, H)), 0.0)
    a_d = jnp.where(blk, jnp.tile(ad1_ref[...], (1, H)), 0.0)
    dn = (((1,), (1,)), ((), ()))
    asrc = lax.dot_general(h, a_s, dn, preferred_element_type=_F32)
    adst = lax.dot_general(h, a_d, dn, preferred_element_type=_F32)
    asrc_ref[...] = asrc
    adst_ref[...] = adst
    asrct_ref[...] = lax.dot_general(a_s, h, dn, preferred_element_type=_F32)
    adstt_ref[...] = lax.dot_general(a_d, h, dn, preferred_element_type=_F32)
    cm = (jnp.max(asrc, axis=0, keepdims=True)
          + jnp.max(adst, axis=0, keepdims=True))
    c = jnp.where(cm >= 0, cm, SLOPE * cm)
    c_ref[...] = jnp.concatenate([c, jnp.zeros((1, L - H), _F32)], axis=1)


def _proj_call(x, w1, as1, ad1):
    return pl.pallas_call(
        _proj_body,
        out_shape=(
            jax.ShapeDtypeStruct((N, CW), _F32),
            jax.ShapeDtypeStruct((N, H), _F32),
            jax.ShapeDtypeStruct((N, H), _F32),
            jax.ShapeDtypeStruct((H, N), _F32),
            jax.ShapeDtypeStruct((H, N), _F32),
            jax.ShapeDtypeStruct((1, L), _F32),
        ),
    )(x, w1, as1, ad1)


def _edge_body(src_h, dst_h, ast_h, adt_h, h_h, c_h,
               p0_h, p1_h,
               ast_v, adt_v, c_v,
               ids_s0, ids_s1, ids_d0, ids_d1, sc_i0, sc_i1,
               hrows0, hrows1, msg0, msg1, acc,
               gsem, isem0, isem1, ssem0, ssem1):
    cid = lax.axis_index("c")
    sid = lax.axis_index("s")
    wid = sid * NC + cid
    bufs = [(ids_s0, ids_d0, sc_i0, hrows0, msg0, isem0, ssem0),
            (ids_s1, ids_d1, sc_i1, hrows1, msg1, isem1, ssem1)]

    pltpu.sync_copy(ast_h, ast_v)
    pltpu.sync_copy(adt_h, adt_v)
    pltpu.sync_copy(c_h, c_v)

    def _zero(i, carry):
        for k in range(AW // L):
            msg0[i, pl.ds(k * L, L)] = jnp.zeros((L,), _F32)
            msg1[i, pl.ds(k * L, L)] = jnp.zeros((L,), _F32)
        return carry
    lax.fori_loop(0, EB, _zero, 0)
    for t in range(ROWS // EB):
        pltpu.sync_copy(msg0, acc.at[pl.ds(sid * ROWS + t * EB, EB)])
    plsc.subcore_barrier()

    cvec = c_v[...]
    cb = [cvec.at[jnp.full((L,), hh, jnp.int32)].get(mode="promise_in_bounds")
          for hh in range(H)]
    lane = jnp.arange(L, dtype=jnp.int32)
    base = wid * EPW

    def _compute(ids_sr, ids_dr, hrowsr, msgr, sc_ir):
        for k in range(EB // L):
            sc_ir[pl.ds(k * L, L)] = ids_dr[pl.ds(k * L, L)]

        def _group(g, carry2):
            sv = ids_sr[pl.ds(g * L, L)]
            dv = ids_dr[pl.ds(g * L, L)]
            ee = []
            for hh in range(H):
                off_h = jnp.int32(hh * N)
                e = (plsc.load_gather(ast_v, [sv + off_h])
                     + plsc.load_gather(adt_v, [dv + off_h]))
                e = jnp.where(e >= 0, e, SLOPE * e)
                ee.append(jnp.exp(e - cb[hh]))
            row0 = g * L
            for hh in range(H):
                plsc.store_scatter(
                    msgr, [row0 + lane, jnp.full((L,), CW + hh, jnp.int32)],
                    ee[hh])
            for j in range(L):
                row = row0 + j
                jv = jnp.full((L,), j, jnp.int32)
                for hh in range(H):
                    bc = ee[hh].at[jv].get(mode="promise_in_bounds")
                    msgr[row, pl.ds(hh * HID, HID)] = (
                        hrowsr[row, pl.ds(hh * HID, HID)] * bc)
            return carry2
        lax.fori_loop(0, EB // L, _group, 0)

    pltpu.sync_copy(src_h.at[pl.ds(base, EB)], ids_s0)
    pltpu.sync_copy(dst_h.at[pl.ds(base, EB)], ids_d0)
    pltpu.async_copy(h_h.at[ids_s0], hrows0, gsem)
    pltpu.async_copy(src_h.at[pl.ds(base + EB, EB)], ids_s1, isem1)
    pltpu.async_copy(dst_h.at[pl.ds(base + EB, EB)], ids_d1, isem1)

    def _half(x, b, i):
        ids_sr, ids_dr, sc_ir, hrowsr, msgr, isem, ssem = bufs[x]
        o_ids_sr, o_ids_dr, _, o_hrowsr, _, o_isem, _ = bufs[1 - x]
        pltpu.make_async_copy(h_h.at[ids_sr], hrowsr, gsem).wait()

        @pl.when(i > 0)
        def _():
            pltpu.make_async_copy(msgr, acc.at[sc_ir], ssem).wait()

        def _next_gather():
            pltpu.make_async_copy(src_h.at[pl.ds(0, EB)], o_ids_sr,
                                  o_isem).wait()
            pltpu.make_async_copy(dst_h.at[pl.ds(0, EB)], o_ids_dr,
                                  o_isem).wait()
            pltpu.async_copy(h_h.at[o_ids_sr], o_hrowsr, gsem)

        if x == 0:
            _next_gather()
        else:
            pl.when(i < NBLK // 2 - 1)(_next_gather)

        _compute(ids_sr, ids_dr, hrowsr, msgr, sc_ir)
        pltpu.async_copy(msgr, acc.at[sc_ir], ssem, add=True)

        @pl.when(i < NBLK // 2 - 1)
        def _():
            off2 = base + (b + 2) * EB
            pltpu.async_copy(src_h.at[pl.ds(off2, EB)], ids_sr, isem)
            pltpu.async_copy(dst_h.at[pl.ds(off2, EB)], ids_dr, isem)

    def _body(i, carry):
        _half(0, 2 * i, i)
        _half(1, 2 * i + 1, i)
        return carry
    lax.fori_loop(0, NBLK // 2, _body, 0)

    pltpu.make_async_copy(msg0, acc.at[sc_i0], ssem0).wait()
    pltpu.make_async_copy(msg1, acc.at[sc_i1], ssem1).wait()


    plsc.subcore_barrier()
    stripe = pl.ds(sid * ROWS, ROWS)

    @pl.when(cid == 0)
    def _():
        pltpu.sync_copy(acc.at[stripe], p0_h.at[stripe])

    @pl.when(cid == 1)
    def _():
        pltpu.sync_copy(acc.at[stripe], p1_h.at[stripe])


def _edge_call(src, dst, asrct, adstt, h, cvec):
    fn = pl.kernel(
        _edge_body,
        out_type=(
            jax.ShapeDtypeStruct((N, AW), _F32),
            jax.ShapeDtypeStruct((N, AW), _F32),
        ),
        mesh=plsc.VectorSubcoreMesh(core_axis_name="c", subcore_axis_name="s",
                                    num_cores=NC, num_subcores=NS),
        compiler_params=pltpu.CompilerParams(needs_layout_passes=False,
                                             use_tc_tiling_on_sc=False),
        scratch_types=[
            pltpu.VMEM((H * N,), _F32),
            pltpu.VMEM((H * N,), _F32),
            pltpu.VMEM((L,), _F32),
            pltpu.VMEM((EB,), jnp.int32),
            pltpu.VMEM((EB,), jnp.int32),
            pltpu.VMEM((EB,), jnp.int32),
            pltpu.VMEM((EB,), jnp.int32),
            pltpu.VMEM((EB,), jnp.int32),
            pltpu.VMEM((EB,), jnp.int32),
            pltpu.VMEM((EB, CW), _F32),
            pltpu.VMEM((EB, CW), _F32),
            pltpu.VMEM((EB, AW), _F32),
            pltpu.VMEM((EB, AW), _F32),
            pltpu.VMEM_SHARED((N, AW), _F32),
            pltpu.SemaphoreType.DMA,
            pltpu.SemaphoreType.DMA,
            pltpu.SemaphoreType.DMA,
            pltpu.SemaphoreType.DMA,
            pltpu.SemaphoreType.DMA,
        ],
    )
    return fn(src, dst, asrct.reshape(H * N), adstt.reshape(H * N), h, cvec)


def _cd_body(p0_ref, p1_ref, h_ref, asrc_ref, adst_ref, c_ref, b1_ref,
             w2_ref, as2_ref, ad2_ref, b2_ref, out_ref,
             amat_s, bmat_s, ae_s, be_s, hh_s, mb_s):
    i = pl.program_id(0)

    @pl.when(i == 0)
    def _():
        c4 = c_ref[0:1, 0:H]
        es = asrc_ref[...] + adst_ref[...]
        es = jnp.where(es >= 0, es, SLOPE * es)
        eself = jnp.exp(es - c4)
        den4 = p0_ref[:, CW:CW + H] + p1_ref[:, CW:CW + H] + eself
        k4 = _head_expand()
        denw = jnp.dot(den4, k4, preferred_element_type=_F32)
        eselfw = jnp.dot(eself, k4, preferred_element_type=_F32)
        num = p0_ref[:, 0:CW] + p1_ref[:, 0:CW] + eselfw * h_ref[...]
        o = num / (denw + 1e-16) + b1_ref[...]
        h1 = jnp.where(o > 0, o, jnp.exp(jnp.minimum(o, 0.0)) - 1.0)
        hh = jnp.dot(h1, w2_ref[...], preferred_element_type=_F32)
        hh_s[...] = hh
        sq = jnp.sum(h1 * h1, axis=1, keepdims=True)
        gs = jnp.dot(hh, as2_ref[...], preferred_element_type=_F32)
        gd = jnp.dot(hh, ad2_ref[...], preferred_element_type=_F32)
        ones_n = jnp.ones((N, 1), _F32)
        amat_s[...] = jnp.concatenate([sq, ones_n, -2.0 * h1], axis=1)
        bmat_s[...] = jnp.concatenate([ones_n, sq, h1], axis=1)
        ae_s[...] = jnp.concatenate([gd, ones_n], axis=1)
        be_s[...] = jnp.concatenate([ones_n, gs], axis=1)
        mb_s[...] = (jnp.max(gd, axis=0, keepdims=True)
                     + jnp.max(gs, axis=0, keepdims=True))

    @pl.when(i > 0)
    def _():
        dn = (((1,), (1,)), ((), ()))
        r0 = (i - 1) * RD
        d2 = lax.dot_general(amat_s[pl.ds(r0, RD), :], bmat_s[...], dn,
                             preferred_element_type=_F32)
        er = lax.dot_general(ae_s[pl.ds(r0, RD), :], be_s[...], dn,
                             preferred_element_type=_F32)
        mbq = mb_s[0, 0]
        mb = jnp.where(mbq >= 0, mbq, SLOPE * mbq)
        e = jnp.maximum(er, SLOPE * er)
        p = jnp.where(d2 < THRESH2, jnp.exp(e - mb), 0.0)
        s = jnp.sum(p, axis=1, keepdims=True)
        o = (jnp.dot(p, hh_s[...], preferred_element_type=_F32) / s
             + b2_ref[...])
        z = o - jnp.max(o, axis=1, keepdims=True)
        out_ref[...] = z - jnp.log(jnp.sum(jnp.exp(z), axis=1,
                                           keepdims=True))


def _cd_call(p0, p1, h, asrc, adst, cpad, b1, w2, as2, ad2, b2):
    full = lambda i: (0, 0)
    return pl.pallas_call(
        _cd_body,
        grid=(GRID + 1,),
        in_specs=[
            pl.BlockSpec((N, AW), full),
            pl.BlockSpec((N, AW), full),
            pl.BlockSpec((N, CW), full),
            pl.BlockSpec((N, H), full),
            pl.BlockSpec((N, H), full),
            pl.BlockSpec((1, L), full),
            pl.BlockSpec((1, CW), full),
            pl.BlockSpec((CW, OUT), full),
            pl.BlockSpec((OUT, 1), full),
            pl.BlockSpec((OUT, 1), full),
            pl.BlockSpec((1, OUT), full),
        ],
        out_specs=pl.BlockSpec((RD, OUT),
                               lambda i: (jnp.maximum(i - 1, 0), 0)),
        out_shape=jax.ShapeDtypeStruct((N, OUT), _F32),
        scratch_shapes=[
            pltpu.VMEM((N, CW + 2), _F32),
            pltpu.VMEM((N, CW + 2), _F32),
            pltpu.VMEM((N, 2), _F32),
            pltpu.VMEM((N, 2), _F32),
            pltpu.VMEM((N, OUT), _F32),
            pltpu.VMEM((1, 1), _F32),
        ],
    )(p0, p1, h, asrc, adst, cpad, b1, w2, as2, ad2, b2)


@jax.jit
def kernel(x, edge_index, W1, a_src1, a_dst1, b1, W2, a_src2, a_dst2, b2):
    src = edge_index[0]
    dst = edge_index[1]
    h, asrc, adst, asrct, adstt, cpad = _proj_call(x, W1, a_src1, a_dst1)
    p0, p1 = _edge_call(src, dst, asrct, adstt, h, cpad.reshape(L))
    return _cd_call(p0, p1, h, asrc, adst, cpad, b1.reshape(1, CW), W2,
                    a_src2.reshape(OUT, 1), a_dst2.reshape(OUT, 1),
                    b2.reshape(1, OUT))


# speedup vs baseline: 62.8608x; 1.0306x over previous
import functools

import jax
import jax.numpy as jnp
from jax import lax
from jax.experimental import pallas as pl
from jax.experimental.pallas import tpu as pltpu
from jax.experimental.pallas import tpu_sc as plsc

N = 4096
E = 65536
IN = 128
HID = 16
H = 4
OUT = 16
THRESH2 = 0.25
SLOPE = 0.2

CW = H * HID
L = 16
AW = CW + L
NC, NS = 2, 16
NW = NC * NS
EPW = E // NW
EB = 128
NBLK = EPW // EB
ROWS = N // NS
RD = 512
GRID = N // RD

_F32 = jnp.float32


def _head_expand():
    heads = lax.broadcasted_iota(jnp.int32, (H, CW), 0)
    chans = lax.broadcasted_iota(jnp.int32, (H, CW), 1)
    return jnp.where((chans // HID) == heads, 1.0, 0.0).astype(_F32)


def _proj_body(x_ref, w1_ref, as1_ref, ad1_ref,
               h_ref, asrc_ref, adst_ref, asrct_ref, adstt_ref, c_ref):
    h = jnp.dot(x_ref[...], w1_ref[...], preferred_element_type=_F32)
    h_ref[...] = h
    heads = lax.broadcasted_iota(jnp.int32, (H, CW), 0)
    chans = lax.broadcasted_iota(jnp.int32, (H, CW), 1)
    blk = (chans // HID) == heads
    a_s = jnp.where(blk, jnp.tile(as1_ref[...], (1, H)), 0.0)
    a_d = jnp.where(blk, jnp.tile(ad1_ref[...], (1, H)), 0.0)
    dn = (((1,), (1,)), ((), ()))
    asrc = lax.dot_general(h, a_s, dn, preferred_element_type=_F32)
    adst = lax.dot_general(h, a_d, dn, preferred_element_type=_F32)
    asrc_ref[...] = asrc
    adst_ref[...] = adst
    asrct_ref[...] = lax.dot_general(a_s, h, dn, preferred_element_type=_F32)
    adstt_ref[...] = lax.dot_general(a_d, h, dn, preferred_element_type=_F32)
    cm = (jnp.max(asrc, axis=0, keepdims=True)
          + jnp.max(adst, axis=0, keepdims=True))
    c = jnp.where(cm >= 0, cm, SLOPE * cm)
    c_ref[...] = jnp.concatenate([c, jnp.zeros((1, L - H), _F32)], axis=1)


def _proj_call(x, w1, as1, ad1):
    return pl.pallas_call(
        _proj_body,
        out_shape=(
            jax.ShapeDtypeStruct((N, CW), _F32),
            jax.ShapeDtypeStruct((N, H), _F32),
            jax.ShapeDtypeStruct((N, H), _F32),
            jax.ShapeDtypeStruct((H, N), _F32),
            jax.ShapeDtypeStruct((H, N), _F32),
            jax.ShapeDtypeStruct((1, L), _F32),
        ),
    )(x, w1, as1, ad1)


def _edge_body(src_h, dst_h, ast_h, adt_h, h_h, c_h,
               p0_h, p1_h,
               ast_v, adt_v, c_v,
               ids_s0, ids_s1, ids_d0, ids_d1, sc_i0, sc_i1,
               hrows0, hrows1, msg0, msg1, acc,
               gsem, isem0, isem1, ssem0, ssem1):
    cid = lax.axis_index("c")
    sid = lax.axis_index("s")
    wid = sid * NC + cid
    bufs = [(ids_s0, ids_d0, sc_i0, hrows0, msg0, isem0, ssem0),
            (ids_s1, ids_d1, sc_i1, hrows1, msg1, isem1, ssem1)]

    pltpu.sync_copy(ast_h, ast_v)
    pltpu.sync_copy(adt_h, adt_v)
    pltpu.sync_copy(c_h, c_v)

    def _zero(i, carry):
        for k in range(AW // L):
            msg0[i, pl.ds(k * L, L)] = jnp.zeros((L,), _F32)
            msg1[i, pl.ds(k * L, L)] = jnp.zeros((L,), _F32)
        return carry
    lax.fori_loop(0, EB, _zero, 0)
    for t in range(ROWS // EB):
        pltpu.sync_copy(msg0, acc.at[pl.ds(sid * ROWS + t * EB, EB)])
    plsc.subcore_barrier()

    cvec = c_v[...]
    cb = [cvec.at[jnp.full((L,), hh, jnp.int32)].get(mode="promise_in_bounds")
          for hh in range(H)]
    lane = jnp.arange(L, dtype=jnp.int32)
    base = wid * EPW

    def _compute(ids_sr, ids_dr, hrowsr, msgr, sc_ir):
        for k in range(EB // L):
            sc_ir[pl.ds(k * L, L)] = ids_dr[pl.ds(k * L, L)]

        def _group(g, carry2):
            sv = ids_sr[pl.ds(g * L, L)]
            dv = ids_dr[pl.ds(g * L, L)]
            ee = []
            for hh in range(H):
                off_h = jnp.int32(hh * N)
                e = (plsc.load_gather(ast_v, [sv + off_h])
                     + plsc.load_gather(adt_v, [dv + off_h]))
                e = jnp.where(e >= 0, e, SLOPE * e)
                ee.append(jnp.exp(e - cb[hh]))
            row0 = g * L
            for hh in range(H):
                plsc.store_scatter(
                    msgr, [row0 + lane, jnp.full((L,), CW + hh, jnp.int32)],
                    ee[hh])
            for j in range(L):
                row = row0 + j
                jv = jnp.full((L,), j, jnp.int32)
                for hh in range(H):
                    bc = ee[hh].at[jv].get(mode="promise_in_bounds")
                    msgr[row, pl.ds(hh * HID, HID)] = (
                        hrowsr[row, pl.ds(hh * HID, HID)] * bc)
            return carry2
        lax.fori_loop(0, EB // L, _group, 0)

    pltpu.sync_copy(src_h.at[pl.ds(base, EB)], ids_s0)
    pltpu.sync_copy(dst_h.at[pl.ds(base, EB)], ids_d0)
    pltpu.async_copy(h_h.at[ids_s0], hrows0, gsem)
    pltpu.async_copy(src_h.at[pl.ds(base + EB, EB)], ids_s1, isem1)
    pltpu.async_copy(dst_h.at[pl.ds(base + EB, EB)], ids_d1, isem1)

    def _half(x, b, i):
        ids_sr, ids_dr, sc_ir, hrowsr, msgr, isem, ssem = bufs[x]
        o_ids_sr, o_ids_dr, _, o_hrowsr, _, o_isem, _ = bufs[1 - x]
        pltpu.make_async_copy(h_h.at[ids_sr], hrowsr, gsem).wait()

        @pl.when(i > 0)
        def _():
            pltpu.make_async_copy(msgr, acc.at[sc_ir], ssem).wait()

        def _next_gather():
            pltpu.make_async_copy(src_h.at[pl.ds(0, EB)], o_ids_sr,
                                  o_isem).wait()
            pltpu.make_async_copy(dst_h.at[pl.ds(0, EB)], o_ids_dr,
                                  o_isem).wait()
            pltpu.async_copy(h_h.at[o_ids_sr], o_hrowsr, gsem)

        if x == 0:
            _next_gather()
        else:
            pl.when(i < NBLK // 2 - 1)(_next_gather)

        _compute(ids_sr, ids_dr, hrowsr, msgr, sc_ir)
        pltpu.async_copy(msgr, acc.at[sc_ir], ssem, add=True)

        @pl.when(i < NBLK // 2 - 1)
        def _():
            off2 = base + (b + 2) * EB
            pltpu.async_copy(src_h.at[pl.ds(off2, EB)], ids_sr, isem)
            pltpu.async_copy(dst_h.at[pl.ds(off2, EB)], ids_dr, isem)

    def _body(i, carry):
        _half(0, 2 * i, i)
        _half(1, 2 * i + 1, i)
        return carry
    lax.fori_loop(0, NBLK // 2, _body, 0)

    pltpu.make_async_copy(msg0, acc.at[sc_i0], ssem0).wait()
    pltpu.make_async_copy(msg1, acc.at[sc_i1], ssem1).wait()


    plsc.subcore_barrier()
    stripe = pl.ds(sid * ROWS, ROWS)

    @pl.when(cid == 0)
    def _():
        pltpu.sync_copy(acc.at[stripe], p0_h.at[stripe])

    @pl.when(cid == 1)
    def _():
        pltpu.sync_copy(acc.at[stripe], p1_h.at[stripe])


def _edge_call(src, dst, asrct, adstt, h, cvec):
    fn = pl.kernel(
        _edge_body,
        out_type=(
            jax.ShapeDtypeStruct((N, AW), _F32),
            jax.ShapeDtypeStruct((N, AW), _F32),
        ),
        mesh=plsc.VectorSubcoreMesh(core_axis_name="c", subcore_axis_name="s",
                                    num_cores=NC, num_subcores=NS),
        compiler_params=pltpu.CompilerParams(needs_layout_passes=False,
                                             use_tc_tiling_on_sc=False),
        scratch_types=[
            pltpu.VMEM((H * N,), _F32),
            pltpu.VMEM((H * N,), _F32),
            pltpu.VMEM((L,), _F32),
            pltpu.VMEM((EB,), jnp.int32),
            pltpu.VMEM((EB,), jnp.int32),
            pltpu.VMEM((EB,), jnp.int32),
            pltpu.VMEM((EB,), jnp.int32),
            pltpu.VMEM((EB,), jnp.int32),
            pltpu.VMEM((EB,), jnp.int32),
            pltpu.VMEM((EB, CW), _F32),
            pltpu.VMEM((EB, CW), _F32),
            pltpu.VMEM((EB, AW), _F32),
            pltpu.VMEM((EB, AW), _F32),
            pltpu.VMEM_SHARED((N, AW), _F32),
            pltpu.SemaphoreType.DMA,
            pltpu.SemaphoreType.DMA,
            pltpu.SemaphoreType.DMA,
            pltpu.SemaphoreType.DMA,
            pltpu.SemaphoreType.DMA,
        ],
    )
    return fn(src, dst, asrct.reshape(H * N), adstt.reshape(H * N), h, cvec)


def _cd_body(p0_ref, p1_ref, h_ref, asrc_ref, adst_ref, c_ref, b1_ref,
             w2_ref, as2_ref, ad2_ref, b2_ref, out_ref,
             amat_s, bmat_s, ae_s, be_s, hh_s, mb_s):
    i = pl.program_id(0)

    @pl.when(i == 0)
    def _():
        c4 = c_ref[0:1, 0:H]
        es = asrc_ref[...] + adst_ref[...]
        es = jnp.where(es >= 0, es, SLOPE * es)
        eself = jnp.exp(es - c4)
        den4 = p0_ref[:, CW:CW + H] + p1_ref[:, CW:CW + H] + eself
        k4 = _head_expand()
        denw = jnp.dot(den4, k4, preferred_element_type=_F32)
        eselfw = jnp.dot(eself, k4, preferred_element_type=_F32)
        num = p0_ref[:, 0:CW] + p1_ref[:, 0:CW] + eselfw * h_ref[...]
        o = num / (denw + 1e-16) + b1_ref[...]
        h1 = jnp.where(o > 0, o, jnp.exp(jnp.minimum(o, 0.0)) - 1.0)
        hh = jnp.dot(h1, w2_ref[...], preferred_element_type=_F32)
        hh_s[...] = hh
        sq = jnp.sum(h1 * h1, axis=1, keepdims=True)
        gs = jnp.dot(hh, as2_ref[...], preferred_element_type=_F32)
        gd = jnp.dot(hh, ad2_ref[...], preferred_element_type=_F32)
        ones_n = jnp.ones((N, 1), _F32)
        amat_s[...] = jnp.concatenate([sq, ones_n, -2.0 * h1], axis=1)
        bmat_s[...] = jnp.concatenate([ones_n, sq, h1], axis=1)
        ae_s[...] = jnp.concatenate([gd, ones_n], axis=1)
        be_s[...] = jnp.concatenate([ones_n, gs], axis=1)
        mb_s[...] = (jnp.max(gd, axis=0, keepdims=True)
                     + jnp.max(gs, axis=0, keepdims=True))

    @pl.when(i > 0)
    def _():
        dn = (((1,), (1,)), ((), ()))
        r0 = (i - 1) * RD
        d2 = lax.dot_general(amat_s[pl.ds(r0, RD), :], bmat_s[...], dn,
                             preferred_element_type=_F32)
        er = lax.dot_general(ae_s[pl.ds(r0, RD), :], be_s[...], dn,
                             preferred_element_type=_F32)
        mbq = mb_s[0, 0]
        mb = jnp.where(mbq >= 0, mbq, SLOPE * mbq)
        e = jnp.maximum(er, SLOPE * er)
        p = jnp.where(d2 < THRESH2, jnp.exp(e - mb), 0.0)
        s = jnp.sum(p, axis=1, keepdims=True)
        o = (jnp.dot(p, hh_s[...], preferred_element_type=_F32) / s
             + b2_ref[...])
        z = o - jnp.max(o, axis=1, keepdims=True)
        out_ref[...] = z - jnp.log(jnp.sum(jnp.exp(z), axis=1,
                                           keepdims=True))


def _cd_call(p0, p1, h, asrc, adst, cpad, b1, w2, as2, ad2, b2):
    full = lambda i: (0, 0)
    return pl.pallas_call(
        _cd_body,
        grid=(GRID + 1,),
        in_specs=[
            pl.BlockSpec((N, AW), full),
            pl.BlockSpec((N, AW), full),
            pl.BlockSpec((N, CW), full),
            pl.BlockSpec((N, H), full),
            pl.BlockSpec((N, H), full),
            pl.BlockSpec((1, L), full),
            pl.BlockSpec((1, CW), full),
            pl.BlockSpec((CW, OUT), full),
            pl.BlockSpec((OUT, 1), full),
            pl.BlockSpec((OUT, 1), full),
            pl.BlockSpec((1, OUT), full),
        ],
        out_specs=pl.BlockSpec((RD, OUT),
                               lambda i: (jnp.maximum(i - 1, 0), 0)),
        out_shape=jax.ShapeDtypeStruct((N, OUT), _F32),
        scratch_shapes=[
            pltpu.VMEM((N, CW + 2), _F32),
            pltpu.VMEM((N, CW + 2), _F32),
            pltpu.VMEM((N, 2), _F32),
            pltpu.VMEM((N, 2), _F32),
            pltpu.VMEM((N, OUT), _F32),
            pltpu.VMEM((1, 1), _F32),
        ],
    )(p0, p1, h, asrc, adst, cpad, b1, w2, as2, ad2, b2)


@jax.jit
def kernel(x, edge_index, W1, a_src1, a_dst1, b1, W2, a_src2, a_dst2, b2):
    src = edge_index[0]
    dst = edge_index[1]
    h, asrc, adst, asrct, adstt, cpad = _proj_call(x, W1, a_src1, a_dst1)
    p0, p1 = _edge_call(src, dst, asrct, adstt, h, cpad.reshape(L))
    return _cd_call(p0, p1, h, asrc, adst, cpad, b1.reshape(1, CW), W2,
                    a_src2.reshape(OUT, 1), a_dst2.reshape(OUT, 1),
                    b2.reshape(1, OUT))


# speedup vs baseline: 71.9667x; 1.1449x over previous
import functools

import jax
import jax.numpy as jnp
from jax import lax
from jax.experimental import pallas as pl
from jax.experimental.pallas import tpu as pltpu
from jax.experimental.pallas import tpu_sc as plsc

N = 4096
E = 65536
IN = 128
HID = 16
H = 4
OUT = 16
THRESH2 = 0.25
SLOPE = 0.2

CW = H * HID
L = 16
AW = CW + L
NC, NS = 2, 16
NW = NC * NS
EPW = E // NW
EB = 128
NBLK = EPW // EB
ROWS = N // NS
RD = 512
GRID = N // RD

_F32 = jnp.float32


def _head_expand():
    heads = lax.broadcasted_iota(jnp.int32, (H, CW), 0)
    chans = lax.broadcasted_iota(jnp.int32, (H, CW), 1)
    return jnp.where((chans // HID) == heads, 1.0, 0.0).astype(_F32)


def _proj_body(x_ref, w1_ref, as1_ref, ad1_ref,
               h_ref, asrc_ref, adst_ref, c_ref):
    h = jnp.dot(x_ref[...], w1_ref[...], preferred_element_type=_F32)
    h_ref[...] = h
    heads = lax.broadcasted_iota(jnp.int32, (H, CW), 0)
    chans = lax.broadcasted_iota(jnp.int32, (H, CW), 1)
    blk = (chans // HID) == heads
    a_s = jnp.where(blk, jnp.tile(as1_ref[...], (1, H)), 0.0)
    a_d = jnp.where(blk, jnp.tile(ad1_ref[...], (1, H)), 0.0)
    dn = (((1,), (1,)), ((), ()))
    asrc = lax.dot_general(h, a_s, dn, preferred_element_type=_F32)
    adst = lax.dot_general(h, a_d, dn, preferred_element_type=_F32)
    asrc_ref[...] = asrc
    adst_ref[...] = adst
    cm = (jnp.max(asrc, axis=0, keepdims=True)
          + jnp.max(adst, axis=0, keepdims=True))
    c = jnp.where(cm >= 0, cm, SLOPE * cm)
    c_ref[...] = jnp.concatenate([c, jnp.zeros((1, L - H), _F32)], axis=1)


def _proj_call(x, w1, as1, ad1):
    return pl.pallas_call(
        _proj_body,
        out_shape=(
            jax.ShapeDtypeStruct((N, CW), _F32),
            jax.ShapeDtypeStruct((N, H), _F32),
            jax.ShapeDtypeStruct((N, H), _F32),
            jax.ShapeDtypeStruct((1, L), _F32),
        ),
    )(x, w1, as1, ad1)


def _edge_body(src_h, dst_h, ast_h, adt_h, h_h, c_h,
               p0_h, p1_h,
               ast_v, adt_v, c_v,
               ids_s0, ids_s1, ids_d0, ids_d1, sc_i0, sc_i1,
               hrows0, hrows1, msg0, msg1, acc,
               gsem, isem0, isem1, ssem0, ssem1):
    cid = lax.axis_index("c")
    sid = lax.axis_index("s")
    wid = sid * NC + cid
    bufs = [(ids_s0, ids_d0, sc_i0, hrows0, msg0, isem0, ssem0),
            (ids_s1, ids_d1, sc_i1, hrows1, msg1, isem1, ssem1)]

    pltpu.sync_copy(ast_h, ast_v)
    pltpu.sync_copy(adt_h, adt_v)
    pltpu.sync_copy(c_h, c_v)

    def _zero(i, carry):
        for k in range(AW // L):
            msg0[i, pl.ds(k * L, L)] = jnp.zeros((L,), _F32)
            msg1[i, pl.ds(k * L, L)] = jnp.zeros((L,), _F32)
        return carry
    lax.fori_loop(0, EB, _zero, 0)
    for t in range(ROWS // EB):
        pltpu.sync_copy(msg0, acc.at[pl.ds(sid * ROWS + t * EB, EB)])
    plsc.subcore_barrier()

    cvec = c_v[...]
    cb = [cvec.at[jnp.full((L,), hh, jnp.int32)].get(mode="promise_in_bounds")
          for hh in range(H)]
    lane = jnp.arange(L, dtype=jnp.int32)
    base = wid * EPW

    def _compute(ids_sr, ids_dr, hrowsr, msgr, sc_ir):
        for k in range(EB // L):
            sc_ir[pl.ds(k * L, L)] = ids_dr[pl.ds(k * L, L)]

        def _group(g, carry2):
            sv = ids_sr[pl.ds(g * L, L)]
            dv = ids_dr[pl.ds(g * L, L)]
            ee = []
            sv4 = sv * jnp.int32(H)
            dv4 = dv * jnp.int32(H)
            for hh in range(H):
                off_h = jnp.int32(hh)
                e = (plsc.load_gather(ast_v, [sv4 + off_h])
                     + plsc.load_gather(adt_v, [dv4 + off_h]))
                e = jnp.where(e >= 0, e, SLOPE * e)
                ee.append(jnp.exp(e - cb[hh]))
            row0 = g * L
            for hh in range(H):
                plsc.store_scatter(
                    msgr, [row0 + lane, jnp.full((L,), CW + hh, jnp.int32)],
                    ee[hh])
            for j in range(L):
                row = row0 + j
                jv = jnp.full((L,), j, jnp.int32)
                for hh in range(H):
                    bc = ee[hh].at[jv].get(mode="promise_in_bounds")
                    msgr[row, pl.ds(hh * HID, HID)] = (
                        hrowsr[row, pl.ds(hh * HID, HID)] * bc)
            return carry2
        for g in range(EB // L):
            _group(g, 0)

    pltpu.sync_copy(src_h.at[pl.ds(base, EB)], ids_s0)
    pltpu.sync_copy(dst_h.at[pl.ds(base, EB)], ids_d0)
    pltpu.async_copy(h_h.at[ids_s0], hrows0, gsem)
    pltpu.async_copy(src_h.at[pl.ds(base + EB, EB)], ids_s1, isem1)
    pltpu.async_copy(dst_h.at[pl.ds(base + EB, EB)], ids_d1, isem1)

    def _half(x, b, i):
        ids_sr, ids_dr, sc_ir, hrowsr, msgr, isem, ssem = bufs[x]
        o_ids_sr, o_ids_dr, _, o_hrowsr, _, o_isem, _ = bufs[1 - x]
        pltpu.make_async_copy(h_h.at[ids_sr], hrowsr, gsem).wait()

        @pl.when(i > 0)
        def _():
            pltpu.make_async_copy(msgr, acc.at[sc_ir], ssem).wait()

        def _next_gather():
            pltpu.make_async_copy(src_h.at[pl.ds(0, EB)], o_ids_sr,
                                  o_isem).wait()
            pltpu.make_async_copy(dst_h.at[pl.ds(0, EB)], o_ids_dr,
                                  o_isem).wait()
            pltpu.async_copy(h_h.at[o_ids_sr], o_hrowsr, gsem)

        if x == 0:
            _next_gather()
        else:
            pl.when(i < NBLK // 2 - 1)(_next_gather)

        _compute(ids_sr, ids_dr, hrowsr, msgr, sc_ir)
        pltpu.async_copy(msgr, acc.at[sc_ir], ssem, add=True)

        @pl.when(i < NBLK // 2 - 1)
        def _():
            off2 = base + (b + 2) * EB
            pltpu.async_copy(src_h.at[pl.ds(off2, EB)], ids_sr, isem)
            pltpu.async_copy(dst_h.at[pl.ds(off2, EB)], ids_dr, isem)

    def _body(i, carry):
        _half(0, 2 * i, i)
        _half(1, 2 * i + 1, i)
        return carry
    lax.fori_loop(0, NBLK // 2, _body, 0)

    pltpu.make_async_copy(msg0, acc.at[sc_i0], ssem0).wait()
    pltpu.make_async_copy(msg1, acc.at[sc_i1], ssem1).wait()


    plsc.subcore_barrier()
    stripe = pl.ds(sid * ROWS, ROWS)

    @pl.when(cid == 0)
    def _():
        pltpu.sync_copy(acc.at[stripe], p0_h.at[stripe])

    @pl.when(cid == 1)
    def _():
        pltpu.sync_copy(acc.at[stripe], p1_h.at[stripe])


def _edge_call(src, dst, asrct, adstt, h, cvec):
    fn = pl.kernel(
        _edge_body,
        out_type=(
            jax.ShapeDtypeStruct((N, AW), _F32),
            jax.ShapeDtypeStruct((N, AW), _F32),
        ),
        mesh=plsc.VectorSubcoreMesh(core_axis_name="c", subcore_axis_name="s",
                                    num_cores=NC, num_subcores=NS),
        compiler_params=pltpu.CompilerParams(needs_layout_passes=False,
                                             use_tc_tiling_on_sc=False),
        scratch_types=[
            pltpu.VMEM((H * N,), _F32),
            pltpu.VMEM((H * N,), _F32),
            pltpu.VMEM((L,), _F32),
            pltpu.VMEM((EB,), jnp.int32),
            pltpu.VMEM((EB,), jnp.int32),
            pltpu.VMEM((EB,), jnp.int32),
            pltpu.VMEM((EB,), jnp.int32),
            pltpu.VMEM((EB,), jnp.int32),
            pltpu.VMEM((EB,), jnp.int32),
            pltpu.VMEM((EB, CW), _F32),
            pltpu.VMEM((EB, CW), _F32),
            pltpu.VMEM((EB, AW), _F32),
            pltpu.VMEM((EB, AW), _F32),
            pltpu.VMEM_SHARED((N, AW), _F32),
            pltpu.SemaphoreType.DMA,
            pltpu.SemaphoreType.DMA,
            pltpu.SemaphoreType.DMA,
            pltpu.SemaphoreType.DMA,
            pltpu.SemaphoreType.DMA,
        ],
    )
    return fn(src, dst, asrct.reshape(N * H), adstt.reshape(N * H), h, cvec)


def _cd_body(p0_ref, p1_ref, h_ref, asrc_ref, adst_ref, c_ref, b1_ref,
             w2_ref, as2_ref, ad2_ref, b2_ref, out_ref,
             amat_s, bmat_s, ae_s, be_s, hh_s, mb_s):
    i = pl.program_id(0)

    @pl.when(i == 0)
    def _():
        c4 = c_ref[0:1, 0:H]
        es = asrc_ref[...] + adst_ref[...]
        es = jnp.where(es >= 0, es, SLOPE * es)
        eself = jnp.exp(es - c4)
        den4 = p0_ref[:, CW:CW + H] + p1_ref[:, CW:CW + H] + eself
        k4 = _head_expand()
        denw = jnp.dot(den4, k4, preferred_element_type=_F32)
        eselfw = jnp.dot(eself, k4, preferred_element_type=_F32)
        num = p0_ref[:, 0:CW] + p1_ref[:, 0:CW] + eselfw * h_ref[...]
        o = num / (denw + 1e-16) + b1_ref[...]
        h1 = jnp.where(o > 0, o, jnp.exp(jnp.minimum(o, 0.0)) - 1.0)
        hh = jnp.dot(h1, w2_ref[...], preferred_element_type=_F32)
        hh_s[...] = hh
        sq = jnp.sum(h1 * h1, axis=1, keepdims=True)
        gs = jnp.dot(hh, as2_ref[...], preferred_element_type=_F32)
        gd = jnp.dot(hh, ad2_ref[...], preferred_element_type=_F32)
        ones_n = jnp.ones((N, 1), _F32)
        amat_s[...] = jnp.concatenate([sq, ones_n, -2.0 * h1], axis=1)
        bmat_s[...] = jnp.concatenate([ones_n, sq, h1], axis=1)
        ae_s[...] = jnp.concatenate([gd, ones_n], axis=1)
        be_s[...] = jnp.concatenate([ones_n, gs], axis=1)
        mb_s[...] = (jnp.max(gd, axis=0, keepdims=True)
                     + jnp.max(gs, axis=0, keepdims=True))

    @pl.when(i > 0)
    def _():
        dn = (((1,), (1,)), ((), ()))
        r0 = (i - 1) * RD
        d2 = lax.dot_general(amat_s[pl.ds(r0, RD), :], bmat_s[...], dn,
                             preferred_element_type=_F32)
        er = lax.dot_general(ae_s[pl.ds(r0, RD), :], be_s[...], dn,
                             preferred_element_type=_F32)
        mbq = mb_s[0, 0]
        mb = jnp.where(mbq >= 0, mbq, SLOPE * mbq)
        e = jnp.maximum(er, SLOPE * er)
        p = jnp.where(d2 < THRESH2, jnp.exp(e - mb), 0.0)
        s = jnp.sum(p, axis=1, keepdims=True)
        o = (jnp.dot(p, hh_s[...], preferred_element_type=_F32) / s
             + b2_ref[...])
        z = o - jnp.max(o, axis=1, keepdims=True)
        out_ref[...] = z - jnp.log(jnp.sum(jnp.exp(z), axis=1,
                                           keepdims=True))


def _cd_call(p0, p1, h, asrc, adst, cpad, b1, w2, as2, ad2, b2):
    full = lambda i: (0, 0)
    return pl.pallas_call(
        _cd_body,
        grid=(GRID + 1,),
        in_specs=[
            pl.BlockSpec((N, AW), full),
            pl.BlockSpec((N, AW), full),
            pl.BlockSpec((N, CW), full),
            pl.BlockSpec((N, H), full),
            pl.BlockSpec((N, H), full),
            pl.BlockSpec((1, L), full),
            pl.BlockSpec((1, CW), full),
            pl.BlockSpec((CW, OUT), full),
            pl.BlockSpec((OUT, 1), full),
            pl.BlockSpec((OUT, 1), full),
            pl.BlockSpec((1, OUT), full),
        ],
        out_specs=pl.BlockSpec((RD, OUT),
                               lambda i: (jnp.maximum(i - 1, 0), 0)),
        out_shape=jax.ShapeDtypeStruct((N, OUT), _F32),
        scratch_shapes=[
            pltpu.VMEM((N, CW + 2), _F32),
            pltpu.VMEM((N, CW + 2), _F32),
            pltpu.VMEM((N, 2), _F32),
            pltpu.VMEM((N, 2), _F32),
            pltpu.VMEM((N, OUT), _F32),
            pltpu.VMEM((1, 1), _F32),
        ],
    )(p0, p1, h, asrc, adst, cpad, b1, w2, as2, ad2, b2)


@jax.jit
def kernel(x, edge_index, W1, a_src1, a_dst1, b1, W2, a_src2, a_dst2, b2):
    src = edge_index[0]
    dst = edge_index[1]
    h, asrc, adst, cpad = _proj_call(x, W1, a_src1, a_dst1)
    p0, p1 = _edge_call(src, dst, asrc, adst, h, cpad.reshape(L))
    return _cd_call(p0, p1, h, asrc, adst, cpad, b1.reshape(1, CW), W2,
                    a_src2.reshape(OUT, 1), a_dst2.reshape(OUT, 1),
                    b2.reshape(1, OUT))
